# baseline (speedup 1.0000x reference)
import functools

import jax
import jax.numpy as jnp
from jax import lax
from jax.experimental import pallas as pl
from jax.experimental.pallas import tpu as pltpu
from jax.experimental.pallas import tpu_sc as plsc

_NUM_CORES = 2
_NUM_SUBCORES = 16
_NUM_WORKERS = _NUM_CORES * _NUM_SUBCORES


@functools.lru_cache(maxsize=None)
def _build(B, D, chunk):
    bpw = B // _NUM_WORKERS
    nch = bpw // chunk
    assert bpw % chunk == 0 and chunk % 8 == 0

    mesh = plsc.VectorSubcoreMesh(core_axis_name="c", subcore_axis_name="s")

    @functools.partial(
        pl.kernel,
        mesh=mesh,
        compiler_params=pltpu.CompilerParams(use_tc_tiling_on_sc=False),
        out_type=jax.ShapeDtypeStruct((B, D), jnp.float32),
        scratch_types=[
            pltpu.VMEM((chunk,), jnp.int32),
            pltpu.VMEM((chunk, D), jnp.float32),
            pltpu.SemaphoreType.DMA,
        ],
    )
    def gather_kernel(table_hbm, idx_hbm, out_hbm, idx_v, rows_v, sem):
        wid = lax.axis_index("s") * _NUM_CORES + lax.axis_index("c")
        base = wid * bpw

        def body(i, carry):
            off = base + i * chunk
            pltpu.sync_copy(idx_hbm.at[pl.ds(off, chunk)], idx_v)
            pltpu.async_copy(table_hbm.at[idx_v], rows_v, sem).wait()
            pltpu.sync_copy(rows_v, out_hbm.at[pl.ds(off, chunk)])
            return carry

        lax.fori_loop(0, nch, body, 0)

    return gather_kernel


@functools.lru_cache(maxsize=None)
def _build_relayout(A, P, C, R, D):
    nblk = A * P * C
    bpw = nblk // _NUM_WORKERS
    assert nblk % _NUM_WORKERS == 0

    mesh = plsc.VectorSubcoreMesh(core_axis_name="c", subcore_axis_name="s")

    @functools.partial(
        pl.kernel,
        mesh=mesh,
        compiler_params=pltpu.CompilerParams(use_tc_tiling_on_sc=True),
        out_type=jax.ShapeDtypeStruct((A, P, C, R, D), jnp.float32),
        scratch_types=[
            pltpu.VMEM((R, D), jnp.float32),
            pltpu.SemaphoreType.DMA,
            pltpu.SemaphoreType.DMA,
        ],
    )
    def relayout_kernel(in_hbm, out_hbm, rows_v, lsem, ssem):
        wid = lax.axis_index("s") * _NUM_CORES + lax.axis_index("c")

        def body(i, carry):
            blk = wid * bpw + i
            a = blk // (P * C)
            p = (blk // C) % P
            c = blk % C
            boff = blk * R * D
            handles = [
                pltpu.async_copy(
                    in_hbm.at[pl.ds(boff + r * D, D)], rows_v.at[r], lsem
                )
                for r in range(R)
            ]
            for h in handles:
                h.wait()
            pltpu.async_copy(rows_v, out_hbm.at[a, p, c], ssem).wait()
            return carry

        lax.fori_loop(0, bpw, body, 0)

    return relayout_kernel


def kernel(word, word_table):
    idx_shape = word.shape
    flat = word.reshape(-1).astype(jnp.int32)
    B = flat.shape[0]
    D = word_table.shape[-1]
    out = _build(B, D, 3000)(word_table, flat)
    A, P, C, R = idx_shape
    return _build_relayout(A, P, C, R, D)(out.reshape(B * D))


# speedup vs baseline: 1.0063x; 1.0063x over previous
import functools

import jax
import jax.numpy as jnp
from jax import lax
from jax.experimental import pallas as pl
from jax.experimental.pallas import tpu as pltpu
from jax.experimental.pallas import tpu_sc as plsc

_NUM_CORES = 2
_NUM_SUBCORES = 16
_NUM_WORKERS = _NUM_CORES * _NUM_SUBCORES


@functools.lru_cache(maxsize=None)
def _build(B, D, chunk):
    bpw = B // _NUM_WORKERS
    nch = bpw // chunk
    assert bpw % chunk == 0 and chunk % 8 == 0

    mesh = plsc.VectorSubcoreMesh(core_axis_name="c", subcore_axis_name="s")

    @functools.partial(
        pl.kernel,
        mesh=mesh,
        compiler_params=pltpu.CompilerParams(use_tc_tiling_on_sc=False),
        out_type=jax.ShapeDtypeStruct((B, D), jnp.float32),
        scratch_types=[
            pltpu.VMEM((chunk,), jnp.int32),
            pltpu.VMEM((chunk, D), jnp.float32),
            pltpu.SemaphoreType.DMA,
        ],
    )
    def gather_kernel(table_hbm, idx_hbm, out_hbm, idx_v, rows_v, sem):
        wid = lax.axis_index("s") * _NUM_CORES + lax.axis_index("c")
        base = wid * bpw

        def body(i, carry):
            off = base + i * chunk
            pltpu.sync_copy(idx_hbm.at[pl.ds(off, chunk)], idx_v)
            pltpu.async_copy(table_hbm.at[idx_v], rows_v, sem).wait()
            pltpu.sync_copy(rows_v, out_hbm.at[pl.ds(off, chunk)])
            return carry

        lax.fori_loop(0, nch, body, 0)

    return gather_kernel


@functools.lru_cache(maxsize=None)
def _build_relayout(A, P, C, R, D):
    nblk = A * P * C
    bpw = nblk // _NUM_WORKERS
    assert nblk % _NUM_WORKERS == 0

    mesh = plsc.VectorSubcoreMesh(core_axis_name="c", subcore_axis_name="s")

    @functools.partial(
        pl.kernel,
        mesh=mesh,
        compiler_params=pltpu.CompilerParams(use_tc_tiling_on_sc=True),
        out_type=jax.ShapeDtypeStruct((A, P, C, R, D), jnp.float32),
        scratch_types=[
            pltpu.VMEM((R * D,), jnp.float32),
            pltpu.VMEM((R, D), jnp.float32),
            pltpu.SemaphoreType.DMA,
            pltpu.SemaphoreType.DMA,
        ],
    )
    def relayout_kernel(in_hbm, out_hbm, flat_v, rows_v, lsem, ssem):
        wid = lax.axis_index("s") * _NUM_CORES + lax.axis_index("c")
        nsub = D // 16

        def body(i, carry):
            blk = wid * bpw + i
            a = blk // (P * C)
            p = (blk // C) % P
            c = blk % C
            boff = blk * R * D
            pltpu.async_copy(
                in_hbm.at[pl.ds(boff, R * D)], flat_v, lsem
            ).wait()
            for r in range(R):
                for s in range(nsub):
                    rows_v[r, pl.ds(s * 16, 16)] = flat_v[
                        pl.ds(r * D + s * 16, 16)
                    ]
            pltpu.async_copy(rows_v, out_hbm.at[a, p, c], ssem).wait()
            return carry

        lax.fori_loop(0, bpw, body, 0)

    return relayout_kernel


def kernel(word, word_table):
    idx_shape = word.shape
    flat = word.reshape(-1).astype(jnp.int32)
    B = flat.shape[0]
    D = word_table.shape[-1]
    out = _build(B, D, 3000)(word_table, flat)
    A, P, C, R = idx_shape
    return _build_relayout(A, P, C, R, D)(out.reshape(B * D))


# speedup vs baseline: 1.7315x; 1.7206x over previous
import functools

import jax
import jax.numpy as jnp
from jax import lax
from jax.experimental import pallas as pl
from jax.experimental.pallas import tpu as pltpu
from jax.experimental.pallas import tpu_sc as plsc

_NUM_CORES = 2
_NUM_SUBCORES = 16
_NUM_WORKERS = _NUM_CORES * _NUM_SUBCORES


@functools.lru_cache(maxsize=None)
def _build(B, D, chunk):
    bpw = B // _NUM_WORKERS
    nch = bpw // chunk
    assert bpw % chunk == 0 and chunk % 8 == 0

    mesh = plsc.VectorSubcoreMesh(core_axis_name="c", subcore_axis_name="s")

    @functools.partial(
        pl.kernel,
        mesh=mesh,
        compiler_params=pltpu.CompilerParams(use_tc_tiling_on_sc=False),
        out_type=jax.ShapeDtypeStruct((B, D), jnp.float32),
        scratch_types=[
            pltpu.VMEM((chunk,), jnp.int32),
            pltpu.VMEM((chunk, D), jnp.float32),
            pltpu.SemaphoreType.DMA,
        ],
    )
    def gather_kernel(table_hbm, idx_hbm, out_hbm, idx_v, rows_v, sem):
        wid = lax.axis_index("s") * _NUM_CORES + lax.axis_index("c")
        base = wid * bpw

        def body(i, carry):
            off = base + i * chunk
            pltpu.sync_copy(idx_hbm.at[pl.ds(off, chunk)], idx_v)
            pltpu.async_copy(table_hbm.at[idx_v], rows_v, sem).wait()
            pltpu.sync_copy(rows_v, out_hbm.at[pl.ds(off, chunk)])
            return carry

        lax.fori_loop(0, nch, body, 0)

    return gather_kernel


def kernel(word, word_table):
    idx_shape = word.shape
    flat = word.reshape(-1).astype(jnp.int32)
    B = flat.shape[0]
    D = word_table.shape[-1]
    out = _build(B, D, 3000)(word_table, flat)
    return out.reshape(idx_shape + (D,))
